# baseline (speedup 1.0000x reference)
import jax
import jax.numpy as jnp
from jax.experimental import pallas as pl
from jax.experimental.pallas import tpu as pltpu

_TILE = 1024


def _fused_body(state_ref, action_ref, values_ref, out_ref):
    s = state_ref[...].astype(jnp.bfloat16)
    v = values_ref[...].astype(jnp.bfloat16)
    q = jnp.dot(s, v, preferred_element_type=jnp.float32)
    out_ref[...] = jnp.sum(action_ref[...] * q, axis=1, keepdims=True)


def kernel(state, action, values):
    batch, state_size = state.shape
    action_size = action.shape[1]
    grid = (batch // _TILE,)
    return pl.pallas_call(
        _fused_body,
        grid=grid,
        in_specs=[
            pl.BlockSpec((_TILE, state_size), lambda i: (i, 0)),
            pl.BlockSpec((_TILE, action_size), lambda i: (i, 0)),
            pl.BlockSpec((state_size, action_size), lambda i: (0, 0)),
        ],
        out_specs=pl.BlockSpec((_TILE, 1), lambda i: (i, 0)),
        out_shape=jax.ShapeDtypeStruct((batch, 1), jnp.float32),
        compiler_params=pltpu.CompilerParams(
            dimension_semantics=("arbitrary",),
        ),
    )(state, action, values)


# speedup vs baseline: 1.0303x; 1.0303x over previous
import jax
import jax.numpy as jnp
from jax.experimental import pallas as pl
from jax.experimental.pallas import tpu as pltpu

_TILE = 2048


def _fused_body(state_ref, action_ref, values_ref, out_ref):
    s = state_ref[...].astype(jnp.bfloat16)
    v = values_ref[...].astype(jnp.bfloat16)
    q = jnp.dot(s, v, preferred_element_type=jnp.float32)
    out_ref[...] = jnp.sum(action_ref[...] * q, axis=1, keepdims=True)


def kernel(state, action, values):
    batch, state_size = state.shape
    action_size = action.shape[1]
    grid = (batch // _TILE,)
    return pl.pallas_call(
        _fused_body,
        grid=grid,
        in_specs=[
            pl.BlockSpec((_TILE, state_size), lambda i: (i, 0)),
            pl.BlockSpec((_TILE, action_size), lambda i: (i, 0)),
            pl.BlockSpec((state_size, action_size), lambda i: (0, 0)),
        ],
        out_specs=pl.BlockSpec((_TILE, 1), lambda i: (i, 0)),
        out_shape=jax.ShapeDtypeStruct((batch, 1), jnp.float32),
        compiler_params=pltpu.CompilerParams(
            dimension_semantics=("arbitrary",),
        ),
    )(state, action, values)
